# baseline (speedup 1.0000x reference)
import jax
import jax.numpy as jnp
from jax.experimental import pallas as pl
from jax.experimental.pallas import tpu as pltpu

_LATENT = 10
_NUM_CLASSES = 10
_TIME_EMB = 32
_TIMESTEPS = 300
_SLAB_K = 64
_X_OFF = 0
_Y_OFF = 16
_T_OFF = 32
_C_ROWS = 16
_TN_ROW = 10
_Y_ROW = 11
_OUT_ROWS = 16


def _round_up(n, m):
    return ((n + m - 1) // m) * m


def _mlp_kernel(c_ref,
                wt1c_ref, bt1c_ref,
                w1p_ref, b1c_ref, w2_ref, b2c_ref,
                w3_ref, b3c_ref, w4p_ref, b4c_ref,
                out_ref, slab_ref):
    f32 = jnp.float32
    bf16 = jnp.bfloat16
    bn = c_ref.shape[1]

    tn_row = c_ref[_TN_ROW:_TN_ROW + 1, :].astype(f32)
    y_row = c_ref[_Y_ROW:_Y_ROW + 1, :].astype(jnp.int32)

    sub16 = jax.lax.broadcasted_iota(jnp.int32, (_C_ROWS, bn), 0)
    slab_ref[_X_OFF:_X_OFF + _C_ROWS, :] = jnp.where(
        sub16 < _LATENT, c_ref[...], jnp.array(0, bf16))
    slab_ref[_Y_OFF:_Y_OFF + _C_ROWS, :] = (sub16 == y_row).astype(bf16)
    slab_ref[_T_OFF:_T_OFF + _TIME_EMB, :] = jnp.maximum(
        wt1c_ref[...] * tn_row + bt1c_ref[...], 0.0).astype(bf16)

    dn = (((0,), (0,)), ((), ()))
    h1 = jnp.maximum(
        jax.lax.dot_general(w1p_ref[...], slab_ref[...], dn,
                            preferred_element_type=f32) + b1c_ref[...], 0.0)
    h2 = jnp.maximum(
        jax.lax.dot_general(w2_ref[...], h1.astype(bf16), dn,
                            preferred_element_type=f32) + b2c_ref[...], 0.0)
    h3 = jnp.maximum(
        jax.lax.dot_general(w3_ref[...], h2.astype(bf16), dn,
                            preferred_element_type=f32) + b3c_ref[...], 0.0)
    out_ref[...] = jax.lax.dot_general(w4p_ref[...], h3.astype(bf16), dn,
                                       preferred_element_type=f32) + b4c_ref[...]


def kernel(emb, wt1, bt1, wt2, bt2, w1, b1, w2, b2, w3, b3, w4, b4, x, t, y):
    f32 = jnp.float32
    bf16 = jnp.bfloat16
    B, latent_dim = x.shape

    block_n = 16384 if B >= 16384 else max(128, _round_up(B, 128))
    Bp = _round_up(B, block_n)

    c = jnp.concatenate([
        x.T,
        (t.astype(f32) / _TIMESTEPS).reshape(1, B),
        y.astype(f32).reshape(1, B),
        jnp.zeros((_C_ROWS - latent_dim - 2, B), f32),
    ]).astype(bf16)
    if Bp != B:
        c = jnp.pad(c, ((0, 0), (0, Bp - B)))

    w1x = w1[:latent_dim]
    w1t = w1[latent_dim:latent_dim + _TIME_EMB]
    w1y = w1[latent_dim + _TIME_EMB:]
    w1p = jnp.zeros((_SLAB_K, w1.shape[1]), f32)
    w1p = w1p.at[_X_OFF:_X_OFF + latent_dim].set(w1x)
    w1p = w1p.at[_Y_OFF:_Y_OFF + _NUM_CLASSES].set(
        jnp.dot(emb, w1y, preferred_element_type=f32))
    w1p = w1p.at[_T_OFF:_T_OFF + _TIME_EMB].set(
        jnp.dot(wt2, w1t, preferred_element_type=f32))
    b1c = (b1 + jnp.dot(bt2, w1t, preferred_element_type=f32)).reshape(-1, 1)
    w4p = jnp.zeros((w4.shape[0], _OUT_ROWS), f32).at[:, :latent_dim].set(w4)
    b4c = jnp.zeros((_OUT_ROWS, 1), f32).at[:latent_dim].set(b4.reshape(-1, 1))

    weights = (wt1.reshape(-1, 1), bt1.reshape(-1, 1),
               w1p.astype(bf16), b1c,
               w2.astype(bf16), b2.reshape(-1, 1),
               w3.astype(bf16), b3.reshape(-1, 1),
               w4p.astype(bf16), b4c)

    VMEM = pltpu.MemorySpace.VMEM

    def const_spec(shape):
        return pl.BlockSpec(shape, lambda i: (0, 0), memory_space=VMEM)

    in_specs = [pl.BlockSpec((_C_ROWS, block_n), lambda i: (0, i),
                             memory_space=VMEM)]
    in_specs += [const_spec(w.shape) for w in weights]

    out_t = pl.pallas_call(
        _mlp_kernel,
        out_shape=jax.ShapeDtypeStruct((_OUT_ROWS, Bp), f32),
        grid=(Bp // block_n,),
        in_specs=in_specs,
        out_specs=pl.BlockSpec((_OUT_ROWS, block_n), lambda i: (0, i),
                               memory_space=VMEM),
        scratch_shapes=[pltpu.VMEM((_SLAB_K, block_n), bf16)],
        compiler_params=pltpu.CompilerParams(
            dimension_semantics=("parallel",)),
    )(c, *weights)
    return out_t[:latent_dim, :B].T


# speedup vs baseline: 1.0136x; 1.0136x over previous
import jax
import jax.numpy as jnp
from jax.experimental import pallas as pl
from jax.experimental.pallas import tpu as pltpu

_LATENT = 10
_NUM_CLASSES = 10
_TIME_EMB = 32
_TIMESTEPS = 300
_SLAB_K = 64
_X_OFF = 0
_Y_OFF = 16
_T_OFF = 32
_C_ROWS = 16
_TN_ROW = 10
_Y_ROW = 11
_OUT_ROWS = 16


def _round_up(n, m):
    return ((n + m - 1) // m) * m


def _mlp_kernel(c_ref,
                wt1c_ref, bt1c_ref,
                w1p_ref, b1c_ref, w2_ref, b2c_ref,
                w3_ref, b3c_ref, w4p_ref, b4c_ref,
                out_ref, slab_ref):
    f32 = jnp.float32
    bf16 = jnp.bfloat16
    bn = c_ref.shape[1]

    tn_row = c_ref[_TN_ROW:_TN_ROW + 1, :]
    y_row = c_ref[_Y_ROW:_Y_ROW + 1, :].astype(jnp.int32)

    sub16 = jax.lax.broadcasted_iota(jnp.int32, (_C_ROWS, bn), 0)
    slab_ref[_X_OFF:_X_OFF + _C_ROWS, :] = jnp.where(
        sub16 < _LATENT, c_ref[...], 0.0).astype(bf16)
    slab_ref[_Y_OFF:_Y_OFF + _C_ROWS, :] = (sub16 == y_row).astype(bf16)
    slab_ref[_T_OFF:_T_OFF + _TIME_EMB, :] = jnp.maximum(
        wt1c_ref[...] * tn_row + bt1c_ref[...], 0.0).astype(bf16)

    dn = (((0,), (0,)), ((), ()))
    h1 = jnp.maximum(
        jax.lax.dot_general(w1p_ref[...], slab_ref[...], dn,
                            preferred_element_type=f32) + b1c_ref[...], 0.0)
    h2 = jnp.maximum(
        jax.lax.dot_general(w2_ref[...], h1.astype(bf16), dn,
                            preferred_element_type=f32) + b2c_ref[...], 0.0)
    h3 = jnp.maximum(
        jax.lax.dot_general(w3_ref[...], h2.astype(bf16), dn,
                            preferred_element_type=f32) + b3c_ref[...], 0.0)
    out_ref[...] = jax.lax.dot_general(w4p_ref[...], h3.astype(bf16), dn,
                                       preferred_element_type=f32) + b4c_ref[...]


def kernel(emb, wt1, bt1, wt2, bt2, w1, b1, w2, b2, w3, b3, w4, b4, x, t, y):
    f32 = jnp.float32
    bf16 = jnp.bfloat16
    B, latent_dim = x.shape

    block_n = 32768 if B >= 32768 else max(128, _round_up(B, 128))
    Bp = _round_up(B, block_n)

    c = jnp.concatenate([
        x.T,
        (t.astype(f32) / _TIMESTEPS).reshape(1, B),
        y.astype(f32).reshape(1, B),
        jnp.zeros((_C_ROWS - latent_dim - 2, B), f32),
    ])
    if Bp != B:
        c = jnp.pad(c, ((0, 0), (0, Bp - B)))

    w1x = w1[:latent_dim]
    w1t = w1[latent_dim:latent_dim + _TIME_EMB]
    w1y = w1[latent_dim + _TIME_EMB:]
    w1p = jnp.zeros((_SLAB_K, w1.shape[1]), f32)
    w1p = w1p.at[_X_OFF:_X_OFF + latent_dim].set(w1x)
    w1p = w1p.at[_Y_OFF:_Y_OFF + _NUM_CLASSES].set(
        jnp.dot(emb, w1y, preferred_element_type=f32))
    w1p = w1p.at[_T_OFF:_T_OFF + _TIME_EMB].set(
        jnp.dot(wt2, w1t, preferred_element_type=f32))
    b1c = (b1 + jnp.dot(bt2, w1t, preferred_element_type=f32)).reshape(-1, 1)
    w4p = jnp.zeros((w4.shape[0], _OUT_ROWS), f32).at[:, :latent_dim].set(w4)
    b4c = jnp.zeros((_OUT_ROWS, 1), f32).at[:latent_dim].set(b4.reshape(-1, 1))

    weights = (wt1.reshape(-1, 1), bt1.reshape(-1, 1),
               w1p.astype(bf16), b1c,
               w2.astype(bf16), b2.reshape(-1, 1),
               w3.astype(bf16), b3.reshape(-1, 1),
               w4p.astype(bf16), b4c)

    VMEM = pltpu.MemorySpace.VMEM

    def const_spec(shape):
        return pl.BlockSpec(shape, lambda i: (0, 0), memory_space=VMEM)

    in_specs = [pl.BlockSpec((_C_ROWS, block_n), lambda i: (0, i),
                             memory_space=VMEM)]
    in_specs += [const_spec(w.shape) for w in weights]

    out_t = pl.pallas_call(
        _mlp_kernel,
        out_shape=jax.ShapeDtypeStruct((_OUT_ROWS, Bp), f32),
        grid=(Bp // block_n,),
        in_specs=in_specs,
        out_specs=pl.BlockSpec((_OUT_ROWS, block_n), lambda i: (0, i),
                               memory_space=VMEM),
        scratch_shapes=[pltpu.VMEM((_SLAB_K, block_n), bf16)],
        compiler_params=pltpu.CompilerParams(
            dimension_semantics=("parallel",)),
    )(c, *weights)
    return out_t[:latent_dim, :B].T


# speedup vs baseline: 1.0708x; 1.0565x over previous
import jax
import jax.numpy as jnp
from jax.experimental import pallas as pl
from jax.experimental.pallas import tpu as pltpu

_LATENT = 10
_NUM_CLASSES = 10
_TIME_EMB = 32
_TIMESTEPS = 300
_SLAB_K = 64
_X_OFF = 0
_Y_OFF = 16
_T_OFF = 32
_C_ROWS = 16
_TN_ROW = 10
_Y_ROW = 11
_OUT_ROWS = 16


def _round_up(n, m):
    return ((n + m - 1) // m) * m


def _mlp_kernel(c_ref,
                wt1c_ref, bt1c_ref,
                w1p_ref, b1c_ref, w2_ref, b2c_ref,
                w3_ref, b3c_ref, w4p_ref, b4c_ref,
                out_ref, slab_ref):
    f32 = jnp.float32
    bf16 = jnp.bfloat16
    bn = c_ref.shape[1]

    tn_row = c_ref[_TN_ROW:_TN_ROW + 1, :].astype(f32)
    y_row = c_ref[_Y_ROW:_Y_ROW + 1, :].astype(jnp.int32)

    sub16 = jax.lax.broadcasted_iota(jnp.int32, (_C_ROWS, bn), 0)
    slab_ref[_X_OFF:_X_OFF + _C_ROWS, :] = jnp.where(
        sub16 < _LATENT, c_ref[...], jnp.array(0, bf16))
    slab_ref[_Y_OFF:_Y_OFF + _C_ROWS, :] = (sub16 == y_row).astype(bf16)
    slab_ref[_T_OFF:_T_OFF + _TIME_EMB, :] = jnp.maximum(
        wt1c_ref[...] * tn_row + bt1c_ref[...], 0.0).astype(bf16)

    dn = (((0,), (0,)), ((), ()))
    h1 = jnp.maximum(
        jax.lax.dot_general(w1p_ref[...], slab_ref[...], dn,
                            preferred_element_type=f32) + b1c_ref[...], 0.0)
    h2 = jnp.maximum(
        jax.lax.dot_general(w2_ref[...], h1.astype(bf16), dn,
                            preferred_element_type=f32) + b2c_ref[...], 0.0)
    h3 = jnp.maximum(
        jax.lax.dot_general(w3_ref[...], h2.astype(bf16), dn,
                            preferred_element_type=f32) + b3c_ref[...], 0.0)
    out_ref[...] = (jax.lax.dot_general(w4p_ref[...], h3.astype(bf16), dn,
                                        preferred_element_type=f32)
                    + b4c_ref[...])[:_LATENT, :]


def kernel(emb, wt1, bt1, wt2, bt2, w1, b1, w2, b2, w3, b3, w4, b4, x, t, y):
    f32 = jnp.float32
    bf16 = jnp.bfloat16
    B, latent_dim = x.shape

    block_n = 32768 if B >= 32768 else max(128, _round_up(B, 128))
    Bp = _round_up(B, block_n)

    c = jnp.concatenate([
        x.T,
        (t.astype(f32) / _TIMESTEPS).reshape(1, B),
        y.astype(f32).reshape(1, B),
        jnp.zeros((_C_ROWS - latent_dim - 2, B), f32),
    ]).astype(bf16)
    if Bp != B:
        c = jnp.pad(c, ((0, 0), (0, Bp - B)))

    w1x = w1[:latent_dim]
    w1t = w1[latent_dim:latent_dim + _TIME_EMB]
    w1y = w1[latent_dim + _TIME_EMB:]
    w1p = jnp.zeros((_SLAB_K, w1.shape[1]), f32)
    w1p = w1p.at[_X_OFF:_X_OFF + latent_dim].set(w1x)
    w1p = w1p.at[_Y_OFF:_Y_OFF + _NUM_CLASSES].set(
        jnp.dot(emb, w1y, preferred_element_type=f32))
    w1p = w1p.at[_T_OFF:_T_OFF + _TIME_EMB].set(
        jnp.dot(wt2, w1t, preferred_element_type=f32))
    b1c = (b1 + jnp.dot(bt2, w1t, preferred_element_type=f32)).reshape(-1, 1)
    w4p = jnp.zeros((w4.shape[0], _OUT_ROWS), f32).at[:, :latent_dim].set(w4)
    b4c = jnp.zeros((_OUT_ROWS, 1), f32).at[:latent_dim].set(b4.reshape(-1, 1))

    weights = (wt1.reshape(-1, 1), bt1.reshape(-1, 1),
               w1p.astype(bf16), b1c,
               w2.astype(bf16), b2.reshape(-1, 1),
               w3.astype(bf16), b3.reshape(-1, 1),
               w4p.astype(bf16), b4c)

    VMEM = pltpu.MemorySpace.VMEM

    def const_spec(shape):
        return pl.BlockSpec(shape, lambda i: (0, 0), memory_space=VMEM)

    in_specs = [pl.BlockSpec((_C_ROWS, block_n), lambda i: (0, i),
                             memory_space=VMEM)]
    in_specs += [const_spec(w.shape) for w in weights]

    out_t = pl.pallas_call(
        _mlp_kernel,
        out_shape=jax.ShapeDtypeStruct((_LATENT, Bp), f32),
        grid=(Bp // block_n,),
        in_specs=in_specs,
        out_specs=pl.BlockSpec((_LATENT, block_n), lambda i: (0, i),
                               memory_space=VMEM),
        scratch_shapes=[pltpu.VMEM((_SLAB_K, block_n), bf16)],
        compiler_params=pltpu.CompilerParams(
            dimension_semantics=("parallel",)),
    )(c, *weights)
    return out_t[:, :B].T
